# baseline (speedup 1.0000x reference)
import functools

import jax
import jax.numpy as jnp
from jax import lax
from jax.experimental import pallas as pl
from jax.experimental.pallas import tpu as pltpu
from jax.experimental.pallas import tpu_sc as plsc

_H = 1024
_W = 1024
_F = 64
_HID = 128
_OUT = 64

_NC = 2
_NS = 16
_NW = _NC * _NS
_LANES = 16

_CH = 128
_GROUPS = _CH // _LANES


def _sc_sample_body(coords_hbm, grid_hbm, out_hbm,
                    coords_v, i00_v, i01_v, i10_v, i11_v, wx_v, wy_v,
                    f00_v, f01_v, f10_v, f11_v, feats_v, sem):
    n_points = coords_hbm.shape[0] // 2
    per_tile = n_points // _NW
    n_chunks = per_tile // _CH
    wid = lax.axis_index("s") * _NC + lax.axis_index("c")
    tile_base = wid * per_tile

    lanes = lax.iota(jnp.int32, _LANES)
    zeros16 = jnp.zeros((_LANES,), jnp.int32)

    def chunk_body(c, carry):
        base = tile_base + c * _CH
        pltpu.sync_copy(coords_hbm.at[pl.ds(2 * base, 2 * _CH)], coords_v)

        for g in range(_GROUPS):
            rows = 2 * (g * _LANES + lanes)
            cx = plsc.load_gather(coords_v, [rows])
            cy = plsc.load_gather(coords_v, [rows + 1])
            xs = cx * jnp.float32(_W - 1)
            ys = cy * jnp.float32(_H - 1)
            x0 = jnp.clip(xs.astype(jnp.int32), 0, _W - 1)
            y0 = jnp.clip(ys.astype(jnp.int32), 0, _H - 1)
            wx = xs - x0.astype(jnp.float32)
            wy = ys - y0.astype(jnp.float32)
            x1 = jnp.minimum(x0 + 1, _W - 1)
            y1 = jnp.minimum(y0 + 1, _H - 1)
            r0 = y0 * _W
            r1 = y1 * _W
            sl = pl.ds(g * _LANES, _LANES)
            i00_v[sl] = r0 + x0
            i01_v[sl] = r0 + x1
            i10_v[sl] = r1 + x0
            i11_v[sl] = r1 + x1
            wx_v[sl] = wx
            wy_v[sl] = wy

        c0 = pltpu.async_copy(grid_hbm.at[i00_v], f00_v, sem)
        c1 = pltpu.async_copy(grid_hbm.at[i01_v], f01_v, sem)
        c2 = pltpu.async_copy(grid_hbm.at[i10_v], f10_v, sem)
        c3 = pltpu.async_copy(grid_hbm.at[i11_v], f11_v, sem)
        c0.wait()
        c1.wait()
        c2.wait()
        c3.wait()

        def pt_body(i, carry2):
            wxs = plsc.load_gather(wx_v, [zeros16 + i])
            wys = plsc.load_gather(wy_v, [zeros16 + i])
            for j in range(_F // _LANES):
                csl = pl.ds(j * _LANES, _LANES)
                a = f00_v[i, csl]
                b = f01_v[i, csl]
                d = f10_v[i, csl]
                e = f11_v[i, csl]
                top = a + wxs * (b - a)
                bot = d + wxs * (e - d)
                feats_v[i, csl] = top + wys * (bot - top)
            return carry2

        lax.fori_loop(0, _CH, pt_body, 0, unroll=False)
        pltpu.sync_copy(feats_v, out_hbm.at[pl.ds(base, _CH)])
        return carry

    lax.fori_loop(0, n_chunks, chunk_body, 0, unroll=False)


def _sc_sample(coords2, grid2):
    n_points = coords2.shape[0] // 2
    return pl.kernel(
        _sc_sample_body,
        out_type=jax.ShapeDtypeStruct((n_points, _F), jnp.float32),
        mesh=plsc.VectorSubcoreMesh(core_axis_name="c", subcore_axis_name="s"),
        compiler_params=pltpu.CompilerParams(
            needs_layout_passes=False, use_tc_tiling_on_sc=False),
        scratch_types=[
            pltpu.VMEM((2 * _CH,), jnp.float32),
            pltpu.VMEM((_CH,), jnp.int32),
            pltpu.VMEM((_CH,), jnp.int32),
            pltpu.VMEM((_CH,), jnp.int32),
            pltpu.VMEM((_CH,), jnp.int32),
            pltpu.VMEM((_CH,), jnp.float32),
            pltpu.VMEM((_CH,), jnp.float32),
            pltpu.VMEM((_CH, _F), jnp.float32),
            pltpu.VMEM((_CH, _F), jnp.float32),
            pltpu.VMEM((_CH, _F), jnp.float32),
            pltpu.VMEM((_CH, _F), jnp.float32),
            pltpu.VMEM((_CH, _F), jnp.float32),
            pltpu.SemaphoreType.DMA,
        ],
    )(coords2, grid2)


def _mlp_body(f_ref, w1_ref, b1_ref, w2_ref, b2_ref, o_ref):
    h = jnp.dot(f_ref[...], w1_ref[...], preferred_element_type=jnp.float32)
    h = jnp.maximum(h + b1_ref[...], 0.0)
    o = jnp.dot(h, w2_ref[...], preferred_element_type=jnp.float32)
    o_ref[...] = o + b2_ref[...]


def _mlp(feats, W1, b1, W2, b2):
    n_points = feats.shape[0]
    bm = 2048
    return pl.pallas_call(
        _mlp_body,
        grid=(n_points // bm,),
        in_specs=[
            pl.BlockSpec((bm, _F), lambda i: (i, 0)),
            pl.BlockSpec((_F, _HID), lambda i: (0, 0)),
            pl.BlockSpec((1, _HID), lambda i: (0, 0)),
            pl.BlockSpec((_HID, _OUT), lambda i: (0, 0)),
            pl.BlockSpec((1, _OUT), lambda i: (0, 0)),
        ],
        out_specs=pl.BlockSpec((bm, _OUT), lambda i: (i, 0)),
        out_shape=jax.ShapeDtypeStruct((n_points, _OUT), jnp.float32),
    )(feats, W1, b1.reshape(1, _HID), W2, b2.reshape(1, _OUT))


def kernel(coords, feature_field, W1, b1, W2, b2):
    shape = coords.shape
    coords2 = coords.reshape(-1)
    grid2 = feature_field.reshape(_H * _W, _F)
    feats = _sc_sample(coords2, grid2)
    out = _mlp(feats, W1, b1, W2, b2)
    return out.reshape(*shape[:-1], _OUT)
